# baseline (speedup 1.0000x reference)
import jax
import jax.numpy as jnp
from jax import lax
from jax.experimental import pallas as pl
from jax.experimental.pallas import tpu as pltpu, tpu_sc as plsc

NC, NS, L = 2, 16, 16
NW = NC * NS
B = 16384
D = 32
NROWS = 1000000
BPW = B // NW
CHUNK = 128
NCHUNK = BPW // CHUNK
NSEG = CHUNK // L


def _collapse_body(W1, b1, W2, b2, W3, b3, Wf, bf, out_ref):
    wf = Wf[...][:, 0]
    t1 = jnp.sum(W3[...] * wf[None, :], axis=1)
    t2 = jnp.sum(W2[...] * t1[None, :], axis=1)
    w = jnp.sum(W1[...] * t2[None, :], axis=1)
    c = (jnp.sum(b1[...] * t2) + jnp.sum(b2[...] * t1)
         + jnp.sum(b3[...] * wf) + bf[...][0])
    out_ref[0:64, :] = jnp.broadcast_to(w[:, None], (64, L))
    out_ref[64:65, :] = jnp.full((1, L), c, jnp.float32)


def _collapse(W1, b1, W2, b2, W3, b3, Wf, bf):
    return pl.pallas_call(
        _collapse_body,
        out_shape=jax.ShapeDtypeStruct((2 * D + 1, L), jnp.float32),
    )(W1, b1, W2, b2, W3, b3, Wf, bf)


def _ncf_body(users_hbm, items_hbm, ut_hbm, it_hbm, wbc_hbm, out_hbm,
              uidx, iidx, idxu0, idxu1, idxi0, idxi1,
              gu0, gu1, gi0, gi1, wbc, accv, outv, su0, su1, si0, si1):
    wid = lax.axis_index("c") * NS + lax.axis_index("s")
    pltpu.sync_copy(wbc_hbm, wbc)
    pltpu.sync_copy(users_hbm.at[wid], uidx)
    pltpu.sync_copy(items_hbm.at[wid], iidx)

    idxu = (idxu0, idxu1)
    idxi = (idxi0, idxi1)
    gu = (gu0, gu1)
    gi = (gi0, gi1)
    sems_u = (su0, su1)
    sems_i = (si0, si1)
    iota = lax.iota(jnp.int32, L)
    cv = wbc[2 * D]

    def build_idx(k, slot):
        for s in range(NSEG):
            seg = pl.ds(s * L, L)
            idxu[slot][seg] = uidx[k, seg]
            idxi[slot][seg] = iidx[k, seg]

    def fire(slot):
        pltpu.async_copy(ut_hbm.at[idxu[slot]], gu[slot], sems_u[slot])
        pltpu.async_copy(it_hbm.at[idxi[slot]], gi[slot], sems_i[slot])

    def drain(slot):
        pltpu.make_async_copy(
            ut_hbm.at[pl.ds(0, CHUNK)], gu[slot], sems_u[slot]).wait()
        pltpu.make_async_copy(
            it_hbm.at[pl.ds(0, CHUNK)], gi[slot], sems_i[slot]).wait()

    def compute(k, slot):
        gus, gis = gu[slot], gi[slot]
        for s in range(NSEG):
            accv[pl.ds(s * L, L)] = cv

        def step(d, carry):
            wu = wbc[d]
            wi = wbc[D + d]
            lane = jnp.full((L,), 0, jnp.int32) + d
            for s in range(NSEG):
                seg = pl.ds(s * L, L)
                rows = s * L + iota
                eu = plsc.load_gather(gus, [rows, lane])
                ei = plsc.load_gather(gis, [rows, lane])
                accv[seg] = accv[seg] + eu * wu + ei * wi
            return carry
        lax.fori_loop(0, D, step, 0)

        for s in range(NSEG):
            seg = pl.ds(s * L, L)
            a = accv[seg]
            outv[pl.ds(k * CHUNK + s * L, L)] = 1.0 / (1.0 + jnp.exp(-a))

    build_idx(0, 0)
    fire(0)
    build_idx(1, 1)
    fire(1)

    @pl.loop(0, NCHUNK, step=2)
    def _chunks(g):
        for b in range(2):
            k = g + b
            drain(b)
            compute(k, b)

            @pl.when(k + 2 < NCHUNK)
            def _prefetch():
                build_idx(k + 2, b)
                fire(b)

    pltpu.sync_copy(outv, out_hbm.at[wid])


_ncf_sc = pl.kernel(
    _ncf_body,
    out_type=jax.ShapeDtypeStruct((NW, BPW), jnp.float32),
    mesh=plsc.VectorSubcoreMesh(core_axis_name="c", subcore_axis_name="s"),
    compiler_params=pltpu.CompilerParams(
        needs_layout_passes=False, use_tc_tiling_on_sc=False),
    scratch_types=[
        pltpu.VMEM((NCHUNK, CHUNK), jnp.int32),
        pltpu.VMEM((NCHUNK, CHUNK), jnp.int32),
        pltpu.VMEM((CHUNK,), jnp.int32),
        pltpu.VMEM((CHUNK,), jnp.int32),
        pltpu.VMEM((CHUNK,), jnp.int32),
        pltpu.VMEM((CHUNK,), jnp.int32),
        pltpu.VMEM((CHUNK, D), jnp.float32),
        pltpu.VMEM((CHUNK, D), jnp.float32),
        pltpu.VMEM((CHUNK, D), jnp.float32),
        pltpu.VMEM((CHUNK, D), jnp.float32),
        pltpu.VMEM((2 * D + 1, L), jnp.float32),
        pltpu.VMEM((CHUNK,), jnp.float32),
        pltpu.VMEM((BPW,), jnp.float32),
        pltpu.SemaphoreType.DMA,
        pltpu.SemaphoreType.DMA,
        pltpu.SemaphoreType.DMA,
        pltpu.SemaphoreType.DMA,
    ],
)


def kernel(users, items, user_table, item_table, W1, b1, W2, b2, W3, b3, Wf, bf):
    wbc = _collapse(W1, b1, W2, b2, W3, b3, Wf, bf)
    u3 = users.reshape(NW, NCHUNK, CHUNK)
    i3 = items.reshape(NW, NCHUNK, CHUNK)
    out = _ncf_sc(u3, i3, user_table, item_table, wbc)
    return out.reshape(B, 1)


# speedup vs baseline: 1.0013x; 1.0013x over previous
import jax
import jax.numpy as jnp
from jax import lax
from jax.experimental import pallas as pl
from jax.experimental.pallas import tpu as pltpu, tpu_sc as plsc

NC, NS, L = 2, 16, 16
NW = NC * NS
B = 16384
D = 32
NROWS = 1000000
RPV = 4
VW = RPV * D
BPW = B // NW
CHUNK = 128
NCHUNK = BPW // CHUNK
NSEG = CHUNK // L


def _collapse_body(W1, b1, W2, b2, W3, b3, Wf, bf, out_ref):
    wf = Wf[...][:, 0]
    t1 = jnp.sum(W3[...] * wf[None, :], axis=1)
    t2 = jnp.sum(W2[...] * t1[None, :], axis=1)
    w = jnp.sum(W1[...] * t2[None, :], axis=1)
    c = (jnp.sum(b1[...] * t2) + jnp.sum(b2[...] * t1)
         + jnp.sum(b3[...] * wf) + bf[...][0])
    out_ref[0:64, :] = jnp.broadcast_to(w[:, None], (64, L))
    out_ref[64:65, :] = jnp.full((1, L), c, jnp.float32)


def _collapse(W1, b1, W2, b2, W3, b3, Wf, bf):
    return pl.pallas_call(
        _collapse_body,
        out_shape=jax.ShapeDtypeStruct((2 * D + 1, L), jnp.float32),
    )(W1, b1, W2, b2, W3, b3, Wf, bf)


def _ncf_body(users_hbm, items_hbm, ut_hbm, it_hbm, wbc_hbm, out_hbm,
              uidx, iidx, idxu0, idxu1, idxi0, idxi1,
              gu0, gu1, gi0, gi1, wbc, outv, su0, su1, si0, si1):
    wid = lax.axis_index("c") * NS + lax.axis_index("s")
    pltpu.sync_copy(wbc_hbm, wbc)
    pltpu.sync_copy(users_hbm.at[wid], uidx)
    pltpu.sync_copy(items_hbm.at[wid], iidx)

    idxu = (idxu0, idxu1)
    idxi = (idxi0, idxi1)
    gu = (gu0, gu1)
    gi = (gi0, gi1)
    sems_u = (su0, su1)
    sems_i = (si0, si1)
    iota = lax.iota(jnp.int32, L)
    cv = wbc[2 * D]

    def build_idx(k, slot):
        for s in range(NSEG):
            uv = uidx[pl.ds(k * CHUNK + s * L, L)]
            iv = iidx[pl.ds(k * CHUNK + s * L, L)]
            idxu[slot][pl.ds(s * L, L)] = lax.shift_right_logical(uv, 2)
            idxi[slot][pl.ds(s * L, L)] = lax.shift_right_logical(iv, 2)

    def fire(slot):
        pltpu.async_copy(ut_hbm.at[idxu[slot]], gu[slot], sems_u[slot])
        pltpu.async_copy(it_hbm.at[idxi[slot]], gi[slot], sems_i[slot])

    def drain(slot):
        pltpu.make_async_copy(
            ut_hbm.at[pl.ds(0, CHUNK)], gu[slot], sems_u[slot]).wait()
        pltpu.make_async_copy(
            it_hbm.at[pl.ds(0, CHUNK)], gi[slot], sems_i[slot]).wait()

    def compute(k, slot):
        gus, gis = gu[slot], gi[slot]
        for s in range(NSEG):
            rows = s * L + iota
            uv = uidx[pl.ds(k * CHUNK + s * L, L)]
            iv = iidx[pl.ds(k * CHUNK + s * L, L)]
            bu = lax.shift_left(jnp.bitwise_and(uv, RPV - 1), 5)
            bi = lax.shift_left(jnp.bitwise_and(iv, RPV - 1), 5)

            def step(d, acc):
                eu = plsc.load_gather(gus, [rows, bu + d])
                ei = plsc.load_gather(gis, [rows, bi + d])
                return acc + eu * wbc[d] + ei * wbc[D + d]

            a = lax.fori_loop(0, D, step, cv)
            outv[pl.ds(k * CHUNK + s * L, L)] = 1.0 / (1.0 + jnp.exp(-a))

    build_idx(0, 0)
    fire(0)
    build_idx(1, 1)
    fire(1)

    @pl.loop(0, NCHUNK, step=2)
    def _chunks(g):
        for b in range(2):
            k = g + b
            drain(b)
            compute(k, b)

            @pl.when(k + 2 < NCHUNK)
            def _prefetch():
                build_idx(k + 2, b)
                fire(b)

    pltpu.sync_copy(outv, out_hbm.at[wid])


_ncf_sc = pl.kernel(
    _ncf_body,
    out_type=jax.ShapeDtypeStruct((NW, BPW), jnp.float32),
    mesh=plsc.VectorSubcoreMesh(core_axis_name="c", subcore_axis_name="s"),
    compiler_params=pltpu.CompilerParams(
        needs_layout_passes=False, use_tc_tiling_on_sc=True),
    scratch_types=[
        pltpu.VMEM((BPW,), jnp.int32),
        pltpu.VMEM((BPW,), jnp.int32),
        pltpu.VMEM((CHUNK,), jnp.int32),
        pltpu.VMEM((CHUNK,), jnp.int32),
        pltpu.VMEM((CHUNK,), jnp.int32),
        pltpu.VMEM((CHUNK,), jnp.int32),
        pltpu.VMEM((CHUNK, VW), jnp.float32),
        pltpu.VMEM((CHUNK, VW), jnp.float32),
        pltpu.VMEM((CHUNK, VW), jnp.float32),
        pltpu.VMEM((CHUNK, VW), jnp.float32),
        pltpu.VMEM((2 * D + 1, L), jnp.float32),
        pltpu.VMEM((BPW,), jnp.float32),
        pltpu.SemaphoreType.DMA,
        pltpu.SemaphoreType.DMA,
        pltpu.SemaphoreType.DMA,
        pltpu.SemaphoreType.DMA,
    ],
)


def kernel(users, items, user_table, item_table, W1, b1, W2, b2, W3, b3, Wf, bf):
    wbc = _collapse(W1, b1, W2, b2, W3, b3, Wf, bf)
    u2 = users.reshape(NW, BPW)
    i2 = items.reshape(NW, BPW)
    ut_v = user_table.reshape(NROWS // RPV, VW)
    it_v = item_table.reshape(NROWS // RPV, VW)
    out = _ncf_sc(u2, i2, ut_v, it_v, wbc)
    return out.reshape(B, 1)


# speedup vs baseline: 1.0126x; 1.0113x over previous
import jax
import jax.numpy as jnp
from jax import lax
from jax.experimental import pallas as pl
from jax.experimental.pallas import tpu as pltpu, tpu_sc as plsc

NC, NS, L = 2, 16, 16
NW = NC * NS
B = 16384
D = 32
NROWS = 1000000
EPR = 16
GPE = D // EPR
BPW = B // NW
CHUNK = 128
NCHUNK = BPW // CHUNK
NSEG = CHUNK // L


def _collapse_body(W1, b1, W2, b2, W3, b3, Wf, bf, out_ref):
    wf = Wf[...][:, 0]
    t1 = jnp.sum(W3[...] * wf[None, :], axis=1)
    t2 = jnp.sum(W2[...] * t1[None, :], axis=1)
    w = jnp.sum(W1[...] * t2[None, :], axis=1)
    c = (jnp.sum(b1[...] * t2) + jnp.sum(b2[...] * t1)
         + jnp.sum(b3[...] * wf) + bf[...][0])
    out_ref[0:64, :] = jnp.broadcast_to(w[:, None], (64, L))
    out_ref[64:65, :] = jnp.full((1, L), c, jnp.float32)


def _collapse(W1, b1, W2, b2, W3, b3, Wf, bf):
    return pl.pallas_call(
        _collapse_body,
        out_shape=jax.ShapeDtypeStruct((2 * D + 1, L), jnp.float32),
    )(W1, b1, W2, b2, W3, b3, Wf, bf)


def _ncf_body(users_hbm, items_hbm, ut_hbm, it_hbm, wbc_hbm, out_hbm,
              uidx, iidx, idxu0, idxu1, idxi0, idxi1,
              gu0, gu1, gi0, gi1, wbc, accv, outv, su0, su1, si0, si1):
    wid = lax.axis_index("c") * NS + lax.axis_index("s")
    pltpu.sync_copy(wbc_hbm, wbc)
    pltpu.sync_copy(users_hbm.at[wid], uidx)
    pltpu.sync_copy(items_hbm.at[wid], iidx)

    idxu = (idxu0, idxu1)
    idxi = (idxi0, idxi1)
    gu = (gu0, gu1)
    gi = (gi0, gi1)
    sems_u = (su0, su1)
    sems_i = (si0, si1)
    iota = lax.iota(jnp.int32, L)
    cv = wbc[2 * D]

    def build_idx(k, slot):
        for s in range(NSEG):
            seg = pl.ds(s * L, L)
            bu = lax.shift_left(uidx[k, seg], 1)
            idxu[slot][0, seg] = bu
            idxu[slot][1, seg] = bu + 1
            bi = lax.shift_left(iidx[k, seg], 1)
            idxi[slot][0, seg] = bi
            idxi[slot][1, seg] = bi + 1

    def fire(slot):
        for j in range(GPE):
            pltpu.async_copy(
                ut_hbm.at[idxu[slot].at[j]],
                gu[slot].at[pl.ds(j * CHUNK, CHUNK)], sems_u[slot])
            pltpu.async_copy(
                it_hbm.at[idxi[slot].at[j]],
                gi[slot].at[pl.ds(j * CHUNK, CHUNK)], sems_i[slot])

    def drain(slot):
        pltpu.make_async_copy(
            ut_hbm.at[pl.ds(0, GPE * CHUNK)], gu[slot], sems_u[slot]).wait()
        pltpu.make_async_copy(
            it_hbm.at[pl.ds(0, GPE * CHUNK)], gi[slot], sems_i[slot]).wait()

    def compute(k, slot):
        gus, gis = gu[slot], gi[slot]
        for s in range(NSEG):
            accv[pl.ds(s * L, L)] = cv

        def step(d, carry):
            wul = wbc[d]
            wuh = wbc[EPR + d]
            wil = wbc[D + d]
            wih = wbc[D + EPR + d]
            lane = jnp.full((L,), 0, jnp.int32) + d
            for s in range(NSEG):
                seg = pl.ds(s * L, L)
                rlo = s * L + iota
                rhi = CHUNK + s * L + iota
                eul = plsc.load_gather(gus, [rlo, lane])
                euh = plsc.load_gather(gus, [rhi, lane])
                eil = plsc.load_gather(gis, [rlo, lane])
                eih = plsc.load_gather(gis, [rhi, lane])
                accv[seg] = (accv[seg] + eul * wul + euh * wuh
                             + eil * wil + eih * wih)
            return carry
        lax.fori_loop(0, EPR, step, 0)

        for s in range(NSEG):
            seg = pl.ds(s * L, L)
            a = accv[seg]
            outv[pl.ds(k * CHUNK + s * L, L)] = 1.0 / (1.0 + jnp.exp(-a))

    build_idx(0, 0)
    fire(0)
    build_idx(1, 1)
    fire(1)

    @pl.loop(0, NCHUNK, step=2)
    def _chunks(g):
        for b in range(2):
            k = g + b
            drain(b)
            compute(k, b)

            @pl.when(k + 2 < NCHUNK)
            def _prefetch():
                build_idx(k + 2, b)
                fire(b)

    pltpu.sync_copy(outv, out_hbm.at[wid])


_ncf_sc = pl.kernel(
    _ncf_body,
    out_type=jax.ShapeDtypeStruct((NW, BPW), jnp.float32),
    mesh=plsc.VectorSubcoreMesh(core_axis_name="c", subcore_axis_name="s"),
    compiler_params=pltpu.CompilerParams(
        needs_layout_passes=False, use_tc_tiling_on_sc=False),
    scratch_types=[
        pltpu.VMEM((NCHUNK, CHUNK), jnp.int32),
        pltpu.VMEM((NCHUNK, CHUNK), jnp.int32),
        pltpu.VMEM((GPE, CHUNK), jnp.int32),
        pltpu.VMEM((GPE, CHUNK), jnp.int32),
        pltpu.VMEM((GPE, CHUNK), jnp.int32),
        pltpu.VMEM((GPE, CHUNK), jnp.int32),
        pltpu.VMEM((GPE * CHUNK, EPR), jnp.float32),
        pltpu.VMEM((GPE * CHUNK, EPR), jnp.float32),
        pltpu.VMEM((GPE * CHUNK, EPR), jnp.float32),
        pltpu.VMEM((GPE * CHUNK, EPR), jnp.float32),
        pltpu.VMEM((2 * D + 1, L), jnp.float32),
        pltpu.VMEM((CHUNK,), jnp.float32),
        pltpu.VMEM((BPW,), jnp.float32),
        pltpu.SemaphoreType.DMA,
        pltpu.SemaphoreType.DMA,
        pltpu.SemaphoreType.DMA,
        pltpu.SemaphoreType.DMA,
    ],
)


def kernel(users, items, user_table, item_table, W1, b1, W2, b2, W3, b3, Wf, bf):
    wbc = _collapse(W1, b1, W2, b2, W3, b3, Wf, bf)
    u3 = users.reshape(NW, NCHUNK, CHUNK)
    i3 = items.reshape(NW, NCHUNK, CHUNK)
    ut_v = user_table.reshape(GPE * NROWS, EPR)
    it_v = item_table.reshape(GPE * NROWS, EPR)
    out = _ncf_sc(u3, i3, ut_v, it_v, wbc)
    return out.reshape(B, 1)
